# baseline (speedup 1.0000x reference)
import functools

import jax
import jax.numpy as jnp
from jax import lax
from jax.experimental import pallas as pl
from jax.experimental.pallas import tpu as pltpu
from jax.experimental.pallas import tpu_sc as plsc

B, T, M = 1, 2048, 1024
E = 8
OUT = 1024
S = B * T
C = 2 * S // E
NSLOT = E * C
NROWS = NSLOT + C
DUMMY = NSLOT

NC, NS = 2, 16
NW = NC * NS
TPW = S // NW
CH = 32
RB = 256


def _ln_router_body(x_ref, wg_ref, xln_ref, log_ref):
    x = x_ref[...]
    mu = jnp.mean(x, axis=-1, keepdims=True)
    xc = x - mu
    var = jnp.mean(xc * xc, axis=-1, keepdims=True)
    xln_ref[...] = xc * lax.rsqrt(var + 1e-5)
    log_ref[...] = jnp.dot(x, wg_ref[...], preferred_element_type=jnp.float32)


def _ln_router(x, wg):
    return pl.pallas_call(
        _ln_router_body,
        grid=(S // RB,),
        in_specs=[
            pl.BlockSpec((RB, M), lambda i: (i, 0)),
            pl.BlockSpec((M, E), lambda i: (0, 0)),
        ],
        out_specs=[
            pl.BlockSpec((RB, M), lambda i: (i, 0)),
            pl.BlockSpec((RB, E), lambda i: (i, 0)),
        ],
        out_shape=[
            jax.ShapeDtypeStruct((S, M), jnp.float32),
            jax.ShapeDtypeStruct((S, E), jnp.float32),
        ],
    )(x, wg)


def _top2_masks(logits):
    iota_e = lax.broadcasted_iota(jnp.int32, logits.shape, 1)
    gates = jax.nn.softmax(logits, axis=-1)
    m1 = jnp.max(logits, axis=-1, keepdims=True)
    idx1 = jnp.min(jnp.where(logits == m1, iota_e, E), axis=-1)
    mask1 = (iota_e == idx1[:, None]).astype(jnp.float32)
    masked = logits + (-1e9) * mask1
    m2 = jnp.max(masked, axis=-1, keepdims=True)
    idx2 = jnp.min(jnp.where(masked == m2, iota_e, E), axis=-1)
    mask2 = (iota_e == idx2[:, None]).astype(jnp.float32)
    return gates, idx1, mask1, idx2, mask2


def _gate_body(logf_ref, logb_ref, sd1_ref, sd2_ref, sc1_ref, sc2_ref,
               g1_ref, g2_ref, laux_ref):
    i = pl.program_id(0)
    logits_f = logf_ref[...]
    gates_f, _, mask1_f, _, mask2_f = _top2_masks(logits_f)
    n1 = jnp.sum(mask1_f, axis=0, keepdims=True)
    laux_ref[...] = (jnp.sum(
        jnp.mean(gates_f, axis=0) * jnp.mean(mask1_f, axis=0)) * E
        ).reshape(1, 1)

    br = lax.broadcasted_iota(jnp.int32, (RB, S), 0) + i * RB
    bc = lax.broadcasted_iota(jnp.int32, (RB, S), 1)
    tri = (br > bc).astype(jnp.float32)
    locs1 = jnp.dot(tri, mask1_f, preferred_element_type=jnp.float32)
    locs2 = jnp.dot(tri, mask2_f, preferred_element_type=jnp.float32) + n1

    logits_b = logb_ref[...]
    gates_b, idx1, mask1, idx2, mask2 = _top2_masks(logits_b)
    mask1c = mask1 * (locs1 < C).astype(jnp.float32)
    mask2c = mask2 * (locs2 < C).astype(jnp.float32)
    loc1 = jnp.sum(locs1 * mask1c, axis=1).astype(jnp.int32)
    loc2 = jnp.sum(locs2 * mask2c, axis=1).astype(jnp.int32)
    gates1 = jnp.sum(gates_b * mask1c, axis=1)
    gates2 = jnp.sum(gates_b * mask2c, axis=1)
    denom = gates1 + gates2
    denom = jnp.where(denom < 1e-9, 1.0, denom)
    g1 = gates1 / denom
    g2 = gates2 / denom
    g1_ref[...] = jnp.broadcast_to(g1[:, None], (RB, 16))
    g2_ref[...] = jnp.broadcast_to(g2[:, None], (RB, 16))
    valid1 = jnp.sum(mask1c, axis=1) > 0
    valid2 = jnp.sum(mask2c, axis=1) > 0
    slot1 = idx1 * C + loc1
    slot2 = idx2 * C + loc2
    sd1_ref[...] = jnp.where(valid1, slot1, DUMMY).reshape(1, 1, RB)
    sd2_ref[...] = jnp.where(valid2, slot2, DUMMY).reshape(1, 1, RB)
    sc1_ref[...] = jnp.where(valid1, slot1, 0).reshape(1, 1, RB)
    sc2_ref[...] = jnp.where(valid2, slot2, 0).reshape(1, 1, RB)


def _gate(logits):
    nblk = S // RB
    islot = jax.ShapeDtypeStruct((nblk, 1, RB), jnp.int32)
    fgate = jax.ShapeDtypeStruct((S, 16), jnp.float32)
    blk3 = pl.BlockSpec((1, 1, RB), lambda i: (i, 0, 0))
    blkg = pl.BlockSpec((RB, 16), lambda i: (i, 0))
    outs = pl.pallas_call(
        _gate_body,
        grid=(nblk,),
        in_specs=[
            pl.BlockSpec((S, E), lambda i: (0, 0)),
            pl.BlockSpec((RB, E), lambda i: (i, 0)),
        ],
        out_specs=[blk3, blk3, blk3, blk3, blkg, blkg,
                   pl.BlockSpec((1, 1), lambda i: (0, 0))],
        out_shape=[islot, islot, islot, islot, fgate, fgate,
                   jax.ShapeDtypeStruct((1, 1), jnp.float32)],
    )(logits, logits)
    sd1, sd2, sc1, sc2, g1, g2, laux = outs
    return (sd1.reshape(S), sd2.reshape(S), sc1.reshape(S), sc2.reshape(S),
            g1, g2, laux)


def _dispatch_body(xln_hbm, sd1_hbm, sd2_hbm, xd_hbm, rows_v, idx_v, sem):
    wid = lax.axis_index("s") * NC + lax.axis_index("c")
    base = wid * TPW
    pltpu.sync_copy(xln_hbm.at[pl.ds(base, TPW)], rows_v)
    pltpu.sync_copy(sd1_hbm.at[pl.ds(base, TPW)], idx_v)
    pltpu.async_copy(rows_v, xd_hbm.at[idx_v], sem).wait()
    pltpu.sync_copy(sd2_hbm.at[pl.ds(base, TPW)], idx_v)
    pltpu.async_copy(rows_v, xd_hbm.at[idx_v], sem).wait()


def _dispatch(x_ln, sd1, sd2):
    mesh = plsc.VectorSubcoreMesh(core_axis_name="c", subcore_axis_name="s")
    f = functools.partial(
        pl.kernel,
        out_type=jax.ShapeDtypeStruct((NROWS, M), jnp.float32),
        mesh=mesh,
        scratch_types=[
            pltpu.VMEM((TPW, M), jnp.float32),
            pltpu.VMEM((TPW,), jnp.int32),
            pltpu.SemaphoreType.DMA,
        ],
    )(_dispatch_body)
    return f(x_ln, sd1, sd2)


def _expert_body(xd_ref, we_ref, lng_ref, lnb_ref, be_ref, y_ref):
    x = xd_ref[...]
    g = lng_ref[...].reshape(1, M)
    b = lnb_ref[...].reshape(1, M)
    normed = x * g + b
    y = jnp.dot(normed, we_ref[0], preferred_element_type=jnp.float32)
    y = y + be_ref[...].reshape(1, OUT)
    y_ref[...] = jax.nn.gelu(y)


def _expert(xd, w_e, ln_g, ln_b, b_e):
    return pl.pallas_call(
        _expert_body,
        grid=(E,),
        in_specs=[
            pl.BlockSpec((C, M), lambda e: (e, 0)),
            pl.BlockSpec((1, M, OUT), lambda e: (e, 0, 0)),
            pl.BlockSpec((1, 1, M), lambda e: (e, 0, 0)),
            pl.BlockSpec((1, 1, M), lambda e: (e, 0, 0)),
            pl.BlockSpec((1, 1, OUT), lambda e: (e, 0, 0)),
        ],
        out_specs=pl.BlockSpec((C, OUT), lambda e: (e, 0)),
        out_shape=jax.ShapeDtypeStruct((NSLOT, OUT), jnp.float32),
    )(xd, w_e, ln_g, ln_b, b_e)


def _combine_accum(rows_v, g_v, out_v, first):
    def body_r(r, _):
        gb = g_v[r]
        valid = gb > 0.0
        zero = jnp.zeros((16,), jnp.float32)

        def body_j(j, _):
            y = rows_v[r, pl.ds(j * 16, 16)]
            contrib = jnp.where(valid, gb * y, zero)
            if first:
                out_v[r, pl.ds(j * 16, 16)] = contrib
            else:
                out_v[r, pl.ds(j * 16, 16)] = out_v[r, pl.ds(j * 16, 16)] + contrib
            return 0

        return lax.fori_loop(0, OUT // 16, body_j, 0)

    lax.fori_loop(0, CH, body_r, 0)


def _combine_body(y_hbm, sc1_hbm, sc2_hbm, g1_hbm, g2_hbm, out_hbm,
                  rows_v, out_v, idx_v, g_v, sem):
    wid = lax.axis_index("s") * NC + lax.axis_index("c")
    for half in range(TPW // CH):
        base = wid * TPW + half * CH
        pltpu.sync_copy(sc1_hbm.at[pl.ds(base, CH)], idx_v)
        pltpu.sync_copy(g1_hbm.at[pl.ds(base, CH)], g_v)
        pltpu.async_copy(y_hbm.at[idx_v], rows_v, sem).wait()
        _combine_accum(rows_v, g_v, out_v, first=True)
        pltpu.sync_copy(sc2_hbm.at[pl.ds(base, CH)], idx_v)
        pltpu.sync_copy(g2_hbm.at[pl.ds(base, CH)], g_v)
        pltpu.async_copy(y_hbm.at[idx_v], rows_v, sem).wait()
        _combine_accum(rows_v, g_v, out_v, first=False)
        pltpu.sync_copy(out_v, out_hbm.at[pl.ds(base, CH)])


def _combine(y, sc1, sc2, g1, g2):
    mesh = plsc.VectorSubcoreMesh(core_axis_name="c", subcore_axis_name="s")
    f = functools.partial(
        pl.kernel,
        out_type=jax.ShapeDtypeStruct((S, OUT), jnp.float32),
        mesh=mesh,
        scratch_types=[
            pltpu.VMEM((CH, OUT), jnp.float32),
            pltpu.VMEM((CH, OUT), jnp.float32),
            pltpu.VMEM((CH,), jnp.int32),
            pltpu.VMEM((CH, 16), jnp.float32),
            pltpu.SemaphoreType.DMA,
        ],
    )(_combine_body)
    return f(y, sc1, sc2, g1, g2)


def kernel(hidden_states, wg, w_e, b_e, ln_g, ln_b):
    x = hidden_states.reshape(S, M)
    x_ln, logits = _ln_router(x, wg)
    sd1, sd2, sc1, sc2, g1, g2, laux = _gate(logits)
    xd = _dispatch(x_ln, sd1, sd2)
    y = _expert(xd, w_e, ln_g.reshape(E, 1, M), ln_b.reshape(E, 1, M),
                b_e.reshape(E, 1, OUT))
    out = _combine(y, sc1, sc2, g1, g2)
    return out.reshape(B, T, OUT), laux.reshape(())


# speedup vs baseline: 1.2418x; 1.2418x over previous
import functools

import jax
import jax.numpy as jnp
from jax import lax
from jax.experimental import pallas as pl
from jax.experimental.pallas import tpu as pltpu
from jax.experimental.pallas import tpu_sc as plsc

B, T, M = 1, 2048, 1024
E = 8
OUT = 1024
S = B * T
C = 2 * S // E
NSLOT = E * C
NROWS = NSLOT + C
DUMMY = NSLOT

NC, NS = 2, 16
NW = NC * NS
TPW = S // NW
CH = 32
RB = 256


def _ln_router_body(x_ref, wg_ref, xln_ref, log_ref):
    x = x_ref[...]
    mu = jnp.mean(x, axis=-1, keepdims=True)
    xc = x - mu
    var = jnp.mean(xc * xc, axis=-1, keepdims=True)
    xln_ref[...] = xc * lax.rsqrt(var + 1e-5)
    log_ref[...] = jnp.dot(x, wg_ref[...], preferred_element_type=jnp.float32)


def _ln_router(x, wg):
    return pl.pallas_call(
        _ln_router_body,
        grid=(S // RB,),
        in_specs=[
            pl.BlockSpec((RB, M), lambda i: (i, 0)),
            pl.BlockSpec((M, E), lambda i: (0, 0)),
        ],
        out_specs=[
            pl.BlockSpec((RB, M), lambda i: (i, 0)),
            pl.BlockSpec((RB, E), lambda i: (i, 0)),
        ],
        out_shape=[
            jax.ShapeDtypeStruct((S, M), jnp.float32),
            jax.ShapeDtypeStruct((S, E), jnp.float32),
        ],
    )(x, wg)


def _top2_masks(logits):
    iota_e = lax.broadcasted_iota(jnp.int32, logits.shape, 1)
    gates = jax.nn.softmax(logits, axis=-1)
    m1 = jnp.max(logits, axis=-1, keepdims=True)
    idx1 = jnp.min(jnp.where(logits == m1, iota_e, E), axis=-1)
    mask1 = (iota_e == idx1[:, None]).astype(jnp.float32)
    masked = logits + (-1e9) * mask1
    m2 = jnp.max(masked, axis=-1, keepdims=True)
    idx2 = jnp.min(jnp.where(masked == m2, iota_e, E), axis=-1)
    mask2 = (iota_e == idx2[:, None]).astype(jnp.float32)
    return gates, idx1, mask1, idx2, mask2


def _gate_body(logf_ref, logb_ref, sd1_ref, sd2_ref, g1_ref, g2_ref,
               laux_ref):
    i = pl.program_id(0)
    logits_f = logf_ref[...]
    gates_f, _, mask1_f, _, mask2_f = _top2_masks(logits_f)
    n1 = jnp.sum(mask1_f, axis=0, keepdims=True)
    laux_ref[...] = (jnp.sum(
        jnp.mean(gates_f, axis=0) * jnp.mean(mask1_f, axis=0)) * E
        ).reshape(1, 1)

    br = lax.broadcasted_iota(jnp.int32, (RB, S), 0) + i * RB
    bc = lax.broadcasted_iota(jnp.int32, (RB, S), 1)
    tri = (br > bc).astype(jnp.float32)
    locs1 = jnp.dot(tri, mask1_f, preferred_element_type=jnp.float32)
    locs2 = jnp.dot(tri, mask2_f, preferred_element_type=jnp.float32) + n1

    logits_b = logb_ref[...]
    gates_b, idx1, mask1, idx2, mask2 = _top2_masks(logits_b)
    mask1c = mask1 * (locs1 < C).astype(jnp.float32)
    mask2c = mask2 * (locs2 < C).astype(jnp.float32)
    loc1 = jnp.sum(locs1 * mask1c, axis=1).astype(jnp.int32)
    loc2 = jnp.sum(locs2 * mask2c, axis=1).astype(jnp.int32)
    gates1 = jnp.sum(gates_b * mask1c, axis=1)
    gates2 = jnp.sum(gates_b * mask2c, axis=1)
    denom = gates1 + gates2
    denom = jnp.where(denom < 1e-9, 1.0, denom)
    g1 = gates1 / denom
    g2 = gates2 / denom
    g1_ref[...] = jnp.broadcast_to(g1[:, None], (RB, 128))
    g2_ref[...] = jnp.broadcast_to(g2[:, None], (RB, 128))
    valid1 = jnp.sum(mask1c, axis=1) > 0
    valid2 = jnp.sum(mask2c, axis=1) > 0
    slot1 = idx1 * C + loc1
    slot2 = idx2 * C + loc2
    sd1_ref[...] = jnp.where(valid1, slot1, DUMMY).reshape(1, 1, RB)
    sd2_ref[...] = jnp.where(valid2, slot2, DUMMY).reshape(1, 1, RB)


def _gate(logits):
    nblk = S // RB
    islot = jax.ShapeDtypeStruct((nblk, 1, RB), jnp.int32)
    fgate = jax.ShapeDtypeStruct((S, 128), jnp.float32)
    blk3 = pl.BlockSpec((1, 1, RB), lambda i: (i, 0, 0))
    blkg = pl.BlockSpec((RB, 128), lambda i: (i, 0))
    outs = pl.pallas_call(
        _gate_body,
        grid=(nblk,),
        in_specs=[
            pl.BlockSpec((S, E), lambda i: (0, 0)),
            pl.BlockSpec((RB, E), lambda i: (i, 0)),
        ],
        out_specs=[blk3, blk3, blkg, blkg,
                   pl.BlockSpec((1, 1), lambda i: (0, 0))],
        out_shape=[islot, islot, fgate, fgate,
                   jax.ShapeDtypeStruct((1, 1), jnp.float32)],
    )(logits, logits)
    sd1, sd2, g1, g2, laux = outs
    return sd1.reshape(S), sd2.reshape(S), g1, g2, laux


def _dispatch_body(xln_hbm, sd1_hbm, sd2_hbm, g1_hbm, g2_hbm,
                   xd_hbm, gslot_hbm, rows_v, g_v, idx_v, sem):
    wid = lax.axis_index("s") * NC + lax.axis_index("c")
    base = wid * TPW
    pltpu.sync_copy(xln_hbm.at[pl.ds(base, TPW)], rows_v)
    pltpu.sync_copy(sd1_hbm.at[pl.ds(base, TPW)], idx_v)
    pltpu.sync_copy(g1_hbm.at[pl.ds(base, TPW)], g_v)
    pltpu.async_copy(rows_v, xd_hbm.at[idx_v], sem).wait()
    pltpu.async_copy(g_v, gslot_hbm.at[idx_v], sem).wait()
    pltpu.sync_copy(sd2_hbm.at[pl.ds(base, TPW)], idx_v)
    pltpu.sync_copy(g2_hbm.at[pl.ds(base, TPW)], g_v)
    pltpu.async_copy(rows_v, xd_hbm.at[idx_v], sem).wait()
    pltpu.async_copy(g_v, gslot_hbm.at[idx_v], sem).wait()


def _dispatch(x_ln, sd1, sd2, g1, g2):
    mesh = plsc.VectorSubcoreMesh(core_axis_name="c", subcore_axis_name="s")
    f = functools.partial(
        pl.kernel,
        out_type=[
            jax.ShapeDtypeStruct((NROWS, M), jnp.float32),
            jax.ShapeDtypeStruct((NROWS, 128), jnp.float32),
        ],
        mesh=mesh,
        scratch_types=[
            pltpu.VMEM((TPW, M), jnp.float32),
            pltpu.VMEM((TPW, 128), jnp.float32),
            pltpu.VMEM((TPW,), jnp.int32),
            pltpu.SemaphoreType.DMA,
        ],
    )(_dispatch_body)
    return f(x_ln, sd1, sd2, g1, g2)


def _expert_body(xd_ref, we_ref, lng_ref, lnb_ref, be_ref, gs_ref, y_ref):
    e = pl.program_id(0)

    @pl.when(e < E)
    def _():
        x = xd_ref[...]
        g = lng_ref[...].reshape(1, M)
        b = lnb_ref[...].reshape(1, M)
        normed = x * g + b
        y = jnp.dot(normed, we_ref[0], preferred_element_type=jnp.float32)
        y = y + be_ref[...].reshape(1, OUT)
        y_ref[...] = jax.nn.gelu(y) * gs_ref[...][:, 0:1]

    @pl.when(e == E)
    def _():
        y_ref[...] = jnp.zeros((C, OUT), jnp.float32)


def _expert(xd, w_e, ln_g, ln_b, b_e, gslot):
    clamp = lambda e: (jnp.minimum(e, E - 1), 0, 0)
    return pl.pallas_call(
        _expert_body,
        grid=(E + 1,),
        in_specs=[
            pl.BlockSpec((C, M), lambda e: (e, 0)),
            pl.BlockSpec((1, M, OUT), clamp),
            pl.BlockSpec((1, 1, M), clamp),
            pl.BlockSpec((1, 1, M), clamp),
            pl.BlockSpec((1, 1, OUT), clamp),
            pl.BlockSpec((C, 128), lambda e: (e, 0)),
        ],
        out_specs=pl.BlockSpec((C, OUT), lambda e: (e, 0)),
        out_shape=jax.ShapeDtypeStruct((NROWS, OUT), jnp.float32),
    )(xd, w_e, ln_g, ln_b, b_e, gslot)


def _combine_body(y_hbm, sd1_hbm, sd2_hbm, out_hbm,
                  rows1_v, rows2_v, idx1_v, idx2_v, sem1, sem2):
    wid = lax.axis_index("s") * NC + lax.axis_index("c")
    for half in range(TPW // CH):
        base = wid * TPW + half * CH
        pltpu.sync_copy(sd1_hbm.at[pl.ds(base, CH)], idx1_v)
        pltpu.sync_copy(sd2_hbm.at[pl.ds(base, CH)], idx2_v)
        cp1 = pltpu.async_copy(y_hbm.at[idx1_v], rows1_v, sem1)
        cp2 = pltpu.async_copy(y_hbm.at[idx2_v], rows2_v, sem2)
        cp1.wait()
        cp2.wait()

        def body_r(r, _):
            for j in range(OUT // 16):
                sl = pl.ds(j * 16, 16)
                rows1_v[r, sl] = rows1_v[r, sl] + rows2_v[r, sl]
            return 0

        lax.fori_loop(0, CH, body_r, 0)
        pltpu.sync_copy(rows1_v, out_hbm.at[pl.ds(base, CH)])


def _combine(y, sd1, sd2):
    mesh = plsc.VectorSubcoreMesh(core_axis_name="c", subcore_axis_name="s")
    f = functools.partial(
        pl.kernel,
        out_type=jax.ShapeDtypeStruct((S, OUT), jnp.float32),
        mesh=mesh,
        scratch_types=[
            pltpu.VMEM((CH, OUT), jnp.float32),
            pltpu.VMEM((CH, OUT), jnp.float32),
            pltpu.VMEM((CH,), jnp.int32),
            pltpu.VMEM((CH,), jnp.int32),
            pltpu.SemaphoreType.DMA,
            pltpu.SemaphoreType.DMA,
        ],
    )(_combine_body)
    return f(y, sd1, sd2)


def kernel(hidden_states, wg, w_e, b_e, ln_g, ln_b):
    x = hidden_states.reshape(S, M)
    x_ln, logits = _ln_router(x, wg)
    sd1, sd2, g1, g2, laux = _gate(logits)
    xd, gslot = _dispatch(x_ln, sd1, sd2, g1, g2)
    y = _expert(xd, w_e, ln_g.reshape(E, 1, M), ln_b.reshape(E, 1, M),
                b_e.reshape(E, 1, OUT), gslot)
    out = _combine(y, sd1, sd2)
    return out.reshape(B, T, OUT), laux.reshape(())


# speedup vs baseline: 1.2786x; 1.0296x over previous
import functools

import jax
import jax.numpy as jnp
from jax import lax
from jax.experimental import pallas as pl
from jax.experimental.pallas import tpu as pltpu
from jax.experimental.pallas import tpu_sc as plsc

B, T, M = 1, 2048, 1024
E = 8
OUT = 1024
S = B * T
C = 2 * S // E
NSLOT = E * C
NROWS = NSLOT + C
DUMMY = NSLOT

NC, NS = 2, 16
NW = NC * NS
TPW = S // NW
CH = 32
RB = 256
NBLK = S // RB


def _top2_masks(logits):
    iota_e = lax.broadcasted_iota(jnp.int32, logits.shape, 1)
    gates = jax.nn.softmax(logits, axis=-1)
    m1 = jnp.max(logits, axis=-1, keepdims=True)
    idx1 = jnp.min(jnp.where(logits == m1, iota_e, E), axis=-1)
    mask1 = (iota_e == idx1[:, None]).astype(jnp.float32)
    masked = logits + (-1e9) * mask1
    m2 = jnp.max(masked, axis=-1, keepdims=True)
    idx2 = jnp.min(jnp.where(masked == m2, iota_e, E), axis=-1)
    mask2 = (iota_e == idx2[:, None]).astype(jnp.float32)
    return gates, idx1, mask1, idx2, mask2


def _gate_body(x_ref, wg_ref, sd1_ref, sd2_ref, g1_ref, g2_ref, laux_ref,
               log_s):
    i = pl.program_id(0)

    @pl.when(i < NBLK)
    def _():
        log_s[pl.ds(i * RB, RB), :] = jnp.dot(
            x_ref[...], wg_ref[...], preferred_element_type=jnp.float32)

    @pl.when(i >= NBLK)
    def _():
        ib = i - NBLK
        logits_f = log_s[...]
        gates_f, _, mask1_f, _, mask2_f = _top2_masks(logits_f)
        n1 = jnp.sum(mask1_f, axis=0, keepdims=True)
        laux_ref[...] = (jnp.sum(
            jnp.mean(gates_f, axis=0) * jnp.mean(mask1_f, axis=0)) * E
            ).reshape(1, 1)

        br = lax.broadcasted_iota(jnp.int32, (RB, S), 0) + ib * RB
        bc = lax.broadcasted_iota(jnp.int32, (RB, S), 1)
        tri = (br > bc).astype(jnp.bfloat16)
        locs1 = jnp.dot(tri, mask1_f.astype(jnp.bfloat16),
                        preferred_element_type=jnp.float32)
        locs2 = jnp.dot(tri, mask2_f.astype(jnp.bfloat16),
                        preferred_element_type=jnp.float32) + n1

        logits_b = log_s[pl.ds(ib * RB, RB), :]
        gates_b, idx1, mask1, idx2, mask2 = _top2_masks(logits_b)
        mask1c = mask1 * (locs1 < C).astype(jnp.float32)
        mask2c = mask2 * (locs2 < C).astype(jnp.float32)
        loc1 = jnp.sum(locs1 * mask1c, axis=1).astype(jnp.int32)
        loc2 = jnp.sum(locs2 * mask2c, axis=1).astype(jnp.int32)
        gates1 = jnp.sum(gates_b * mask1c, axis=1)
        gates2 = jnp.sum(gates_b * mask2c, axis=1)
        denom = gates1 + gates2
        denom = jnp.where(denom < 1e-9, 1.0, denom)
        g1 = gates1 / denom
        g2 = gates2 / denom
        g1_ref[...] = jnp.broadcast_to(g1[:, None], (RB, 128))
        g2_ref[...] = jnp.broadcast_to(g2[:, None], (RB, 128))
        valid1 = jnp.sum(mask1c, axis=1) > 0
        valid2 = jnp.sum(mask2c, axis=1) > 0
        slot1 = idx1 * C + loc1
        slot2 = idx2 * C + loc2
        sd1_ref[...] = jnp.where(valid1, slot1, DUMMY).reshape(1, 1, RB)
        sd2_ref[...] = jnp.where(valid2, slot2, DUMMY).reshape(1, 1, RB)


def _gate(x, wg):
    islot = jax.ShapeDtypeStruct((NBLK, 1, RB), jnp.int32)
    fgate = jax.ShapeDtypeStruct((S, 128), jnp.float32)
    phase_b = lambda i: (jnp.maximum(i - NBLK, 0), 0, 0)
    phase_bg = lambda i: (jnp.maximum(i - NBLK, 0), 0)
    outs = pl.pallas_call(
        _gate_body,
        grid=(2 * NBLK,),
        in_specs=[
            pl.BlockSpec((RB, M), lambda i: (jnp.minimum(i, NBLK - 1), 0)),
            pl.BlockSpec((M, E), lambda i: (0, 0)),
        ],
        out_specs=[
            pl.BlockSpec((1, 1, RB), phase_b),
            pl.BlockSpec((1, 1, RB), phase_b),
            pl.BlockSpec((RB, 128), phase_bg),
            pl.BlockSpec((RB, 128), phase_bg),
            pl.BlockSpec((1, 1), lambda i: (0, 0)),
        ],
        out_shape=[islot, islot, fgate, fgate,
                   jax.ShapeDtypeStruct((1, 1), jnp.float32)],
        scratch_shapes=[pltpu.VMEM((S, E), jnp.float32)],
    )(x, wg)
    sd1, sd2, g1, g2, laux = outs
    return sd1.reshape(S), sd2.reshape(S), g1, g2, laux


def _dispatch_body(x_hbm, sd1_hbm, sd2_hbm, g1_hbm, g2_hbm,
                   xd_hbm, gslot_hbm, rows_v, g_v, idx_v, sem):
    wid = lax.axis_index("s") * NC + lax.axis_index("c")
    base = wid * TPW
    pltpu.sync_copy(x_hbm.at[pl.ds(base, TPW)], rows_v)
    pltpu.sync_copy(sd1_hbm.at[pl.ds(base, TPW)], idx_v)
    pltpu.sync_copy(g1_hbm.at[pl.ds(base, TPW)], g_v)
    pltpu.async_copy(rows_v, xd_hbm.at[idx_v], sem).wait()
    pltpu.async_copy(g_v, gslot_hbm.at[idx_v], sem).wait()
    pltpu.sync_copy(sd2_hbm.at[pl.ds(base, TPW)], idx_v)
    pltpu.sync_copy(g2_hbm.at[pl.ds(base, TPW)], g_v)
    pltpu.async_copy(rows_v, xd_hbm.at[idx_v], sem).wait()
    pltpu.async_copy(g_v, gslot_hbm.at[idx_v], sem).wait()


def _dispatch(x, sd1, sd2, g1, g2):
    mesh = plsc.VectorSubcoreMesh(core_axis_name="c", subcore_axis_name="s")
    f = functools.partial(
        pl.kernel,
        out_type=[
            jax.ShapeDtypeStruct((NROWS, M), jnp.float32),
            jax.ShapeDtypeStruct((NROWS, 128), jnp.float32),
        ],
        mesh=mesh,
        scratch_types=[
            pltpu.VMEM((TPW, M), jnp.float32),
            pltpu.VMEM((TPW, 128), jnp.float32),
            pltpu.VMEM((TPW,), jnp.int32),
            pltpu.SemaphoreType.DMA,
        ],
    )(_dispatch_body)
    return f(x, sd1, sd2, g1, g2)


def _expert_body(xd_ref, we_ref, lng_ref, lnb_ref, be_ref, gs_ref, y_ref):
    e = pl.program_id(0)

    @pl.when(e < E)
    def _():
        x = xd_ref[...]
        mu = jnp.mean(x, axis=-1, keepdims=True)
        xc = x - mu
        var = jnp.mean(xc * xc, axis=-1, keepdims=True)
        xln = xc * lax.rsqrt(var + 1e-5)
        g = lng_ref[...].reshape(1, M)
        b = lnb_ref[...].reshape(1, M)
        normed = (xln * g + b).astype(jnp.bfloat16)
        w = we_ref[0].astype(jnp.bfloat16)
        y = jnp.dot(normed, w, preferred_element_type=jnp.float32)
        y = y + be_ref[...].reshape(1, OUT)
        y_ref[...] = jax.nn.gelu(y) * gs_ref[...][:, 0:1]

    @pl.when(e == E)
    def _():
        y_ref[...] = jnp.zeros((C, OUT), jnp.float32)


def _expert(xd, w_e, ln_g, ln_b, b_e, gslot):
    clamp = lambda e: (jnp.minimum(e, E - 1), 0, 0)
    return pl.pallas_call(
        _expert_body,
        grid=(E + 1,),
        in_specs=[
            pl.BlockSpec((C, M), lambda e: (e, 0)),
            pl.BlockSpec((1, M, OUT), clamp),
            pl.BlockSpec((1, 1, M), clamp),
            pl.BlockSpec((1, 1, M), clamp),
            pl.BlockSpec((1, 1, OUT), clamp),
            pl.BlockSpec((C, 128), lambda e: (e, 0)),
        ],
        out_specs=pl.BlockSpec((C, OUT), lambda e: (e, 0)),
        out_shape=jax.ShapeDtypeStruct((NROWS, OUT), jnp.float32),
    )(xd, w_e, ln_g, ln_b, b_e, gslot)


def _combine_body(y_hbm, sd1_hbm, sd2_hbm, out_hbm,
                  rows1_v, rows2_v, idx1_v, idx2_v, sem1, sem2):
    wid = lax.axis_index("s") * NC + lax.axis_index("c")
    for half in range(TPW // CH):
        base = wid * TPW + half * CH
        pltpu.sync_copy(sd1_hbm.at[pl.ds(base, CH)], idx1_v)
        pltpu.sync_copy(sd2_hbm.at[pl.ds(base, CH)], idx2_v)
        cp1 = pltpu.async_copy(y_hbm.at[idx1_v], rows1_v, sem1)
        cp2 = pltpu.async_copy(y_hbm.at[idx2_v], rows2_v, sem2)
        cp1.wait()
        cp2.wait()

        def body_r(r, _):
            for j in range(OUT // 16):
                sl = pl.ds(j * 16, 16)
                rows1_v[r, sl] = rows1_v[r, sl] + rows2_v[r, sl]
            return 0

        lax.fori_loop(0, CH, body_r, 0)
        pltpu.sync_copy(rows1_v, out_hbm.at[pl.ds(base, CH)])


def _combine(y, sd1, sd2):
    mesh = plsc.VectorSubcoreMesh(core_axis_name="c", subcore_axis_name="s")
    f = functools.partial(
        pl.kernel,
        out_type=jax.ShapeDtypeStruct((S, OUT), jnp.float32),
        mesh=mesh,
        scratch_types=[
            pltpu.VMEM((CH, OUT), jnp.float32),
            pltpu.VMEM((CH, OUT), jnp.float32),
            pltpu.VMEM((CH,), jnp.int32),
            pltpu.VMEM((CH,), jnp.int32),
            pltpu.SemaphoreType.DMA,
            pltpu.SemaphoreType.DMA,
        ],
    )(_combine_body)
    return f(y, sd1, sd2)


def kernel(hidden_states, wg, w_e, b_e, ln_g, ln_b):
    x = hidden_states.reshape(S, M)
    sd1, sd2, g1, g2, laux = _gate(x, wg)
    xd, gslot = _dispatch(x, sd1, sd2, g1, g2)
    y = _expert(xd, w_e, ln_g.reshape(E, 1, M), ln_b.reshape(E, 1, M),
                b_e.reshape(E, 1, OUT), gslot)
    out = _combine(y, sd1, sd2)
    return out.reshape(B, T, OUT), laux.reshape(())


# speedup vs baseline: 1.5630x; 1.2224x over previous
import functools

import jax
import jax.numpy as jnp
from jax import lax
from jax.experimental import pallas as pl
from jax.experimental.pallas import tpu as pltpu
from jax.experimental.pallas import tpu_sc as plsc

B, T, M = 1, 2048, 1024
E = 8
OUT = 1024
S = B * T
C = 2 * S // E
NSLOT = E * C
NROWS = NSLOT + C
DUMMY = NSLOT

NC, NS = 2, 16
NW = NC * NS
TPW = S // NW
CH = 32
RB = 256
NBLK = S // RB


def _top2_masks_t(lt):
    iota_e = lax.broadcasted_iota(jnp.int32, lt.shape, 0)
    mx = jnp.max(lt, axis=0, keepdims=True)
    ex = jnp.exp(lt - mx)
    gates = ex / jnp.sum(ex, axis=0, keepdims=True)
    idx1 = jnp.min(jnp.where(lt == mx, iota_e, E), axis=0)
    mask1 = (iota_e == idx1[None, :]).astype(jnp.float32)
    masked = lt + (-1e9) * mask1
    m2 = jnp.max(masked, axis=0, keepdims=True)
    idx2 = jnp.min(jnp.where(masked == m2, iota_e, E), axis=0)
    mask2 = (iota_e == idx2[None, :]).astype(jnp.float32)
    return gates, idx1, mask1, idx2, mask2


def _gate_body(x_ref, wg_ref, sd1_ref, sd2_ref, g1_ref, g2_ref, laux_ref,
               log_s):
    i = pl.program_id(0)

    @pl.when(i < NBLK)
    def _():
        log_s[:, pl.ds(i * RB, RB)] = lax.dot_general(
            wg_ref[...], x_ref[...], (((0,), (1,)), ((), ())),
            preferred_element_type=jnp.float32)

    @pl.when(i >= NBLK)
    def _():
        ib = i - NBLK
        lt = log_s[...]
        gates_f, _, mask1_f, _, mask2_f = _top2_masks_t(lt)
        n1 = jnp.sum(mask1_f, axis=1, keepdims=True)
        laux_ref[...] = (jnp.sum(
            jnp.mean(gates_f, axis=1) * jnp.mean(mask1_f, axis=1)) * E
            ).reshape(1, 1)

        br = lax.broadcasted_iota(jnp.int32, (S, RB), 0)
        bc = lax.broadcasted_iota(jnp.int32, (S, RB), 1) + ib * RB
        tri = (bc > br).astype(jnp.bfloat16)
        locs1 = jnp.dot(mask1_f.astype(jnp.bfloat16), tri,
                        preferred_element_type=jnp.float32)
        locs2 = jnp.dot(mask2_f.astype(jnp.bfloat16), tri,
                        preferred_element_type=jnp.float32) + n1

        ltb = log_s[:, pl.ds(ib * RB, RB)]
        gates_b, idx1, mask1, idx2, mask2 = _top2_masks_t(ltb)
        mask1c = mask1 * (locs1 < C).astype(jnp.float32)
        mask2c = mask2 * (locs2 < C).astype(jnp.float32)
        loc1 = jnp.sum(locs1 * mask1c, axis=0).astype(jnp.int32)
        loc2 = jnp.sum(locs2 * mask2c, axis=0).astype(jnp.int32)
        gates1 = jnp.sum(gates_b * mask1c, axis=0)
        gates2 = jnp.sum(gates_b * mask2c, axis=0)
        denom = gates1 + gates2
        denom = jnp.where(denom < 1e-9, 1.0, denom)
        g1 = gates1 / denom
        g2 = gates2 / denom
        g1_ref[...] = jnp.broadcast_to(g1[:, None], (RB, 128))
        g2_ref[...] = jnp.broadcast_to(g2[:, None], (RB, 128))
        valid1 = jnp.sum(mask1c, axis=0) > 0
        valid2 = jnp.sum(mask2c, axis=0) > 0
        slot1 = idx1 * C + loc1
        slot2 = idx2 * C + loc2
        sd1_ref[...] = jnp.where(valid1, slot1, DUMMY).reshape(1, 1, RB)
        sd2_ref[...] = jnp.where(valid2, slot2, DUMMY).reshape(1, 1, RB)


def _gate(x, wg):
    islot = jax.ShapeDtypeStruct((NBLK, 1, RB), jnp.int32)
    fgate = jax.ShapeDtypeStruct((S, 128), jnp.float32)
    phase_b = lambda i: (jnp.maximum(i - NBLK, 0), 0, 0)
    phase_bg = lambda i: (jnp.maximum(i - NBLK, 0), 0)
    outs = pl.pallas_call(
        _gate_body,
        grid=(2 * NBLK,),
        in_specs=[
            pl.BlockSpec((RB, M), lambda i: (jnp.minimum(i, NBLK - 1), 0)),
            pl.BlockSpec((M, E), lambda i: (0, 0)),
        ],
        out_specs=[
            pl.BlockSpec((1, 1, RB), phase_b),
            pl.BlockSpec((1, 1, RB), phase_b),
            pl.BlockSpec((RB, 128), phase_bg),
            pl.BlockSpec((RB, 128), phase_bg),
            pl.BlockSpec((1, 1), lambda i: (0, 0)),
        ],
        out_shape=[islot, islot, fgate, fgate,
                   jax.ShapeDtypeStruct((1, 1), jnp.float32)],
        scratch_shapes=[pltpu.VMEM((E, S), jnp.float32)],
    )(x, wg)
    sd1, sd2, g1, g2, laux = outs
    return sd1.reshape(S), sd2.reshape(S), g1, g2, laux


def _dispatch_body(x_hbm, sd1_hbm, sd2_hbm, g1_hbm, g2_hbm,
                   xd_hbm, gslot_hbm, rows_v, g_v, idx_v, sem):
    wid = lax.axis_index("s") * NC + lax.axis_index("c")
    base = wid * TPW
    pltpu.sync_copy(x_hbm.at[pl.ds(base, TPW)], rows_v)
    pltpu.sync_copy(sd1_hbm.at[pl.ds(base, TPW)], idx_v)
    pltpu.sync_copy(g1_hbm.at[pl.ds(base, TPW)], g_v)
    pltpu.async_copy(rows_v, xd_hbm.at[idx_v], sem).wait()
    pltpu.async_copy(g_v, gslot_hbm.at[idx_v], sem).wait()
    pltpu.sync_copy(sd2_hbm.at[pl.ds(base, TPW)], idx_v)
    pltpu.sync_copy(g2_hbm.at[pl.ds(base, TPW)], g_v)
    pltpu.async_copy(rows_v, xd_hbm.at[idx_v], sem).wait()
    pltpu.async_copy(g_v, gslot_hbm.at[idx_v], sem).wait()


def _dispatch(x, sd1, sd2, g1, g2):
    mesh = plsc.VectorSubcoreMesh(core_axis_name="c", subcore_axis_name="s")
    f = functools.partial(
        pl.kernel,
        out_type=[
            jax.ShapeDtypeStruct((NROWS, M), jnp.float32),
            jax.ShapeDtypeStruct((NROWS, 128), jnp.float32),
        ],
        mesh=mesh,
        scratch_types=[
            pltpu.VMEM((TPW, M), jnp.float32),
            pltpu.VMEM((TPW, 128), jnp.float32),
            pltpu.VMEM((TPW,), jnp.int32),
            pltpu.SemaphoreType.DMA,
        ],
    )(_dispatch_body)
    return f(x, sd1, sd2, g1, g2)


def _expert_body(xd_ref, we_ref, lng_ref, lnb_ref, be_ref, gs_ref, y_ref):
    e = pl.program_id(0)

    @pl.when(e < E)
    def _():
        x = xd_ref[...]
        mu = jnp.mean(x, axis=-1, keepdims=True)
        xc = x - mu
        var = jnp.mean(xc * xc, axis=-1, keepdims=True)
        xln = xc * lax.rsqrt(var + 1e-5)
        g = lng_ref[...].reshape(1, M)
        b = lnb_ref[...].reshape(1, M)
        normed = (xln * g + b).astype(jnp.bfloat16)
        w = we_ref[0].astype(jnp.bfloat16)
        y = jnp.dot(normed, w, preferred_element_type=jnp.float32)
        y = y + be_ref[...].reshape(1, OUT)
        y_ref[...] = jax.nn.gelu(y) * gs_ref[...][:, 0:1]

    @pl.when(e == E)
    def _():
        y_ref[...] = jnp.zeros((C, OUT), jnp.float32)


def _expert(xd, w_e, ln_g, ln_b, b_e, gslot):
    clamp = lambda e: (jnp.minimum(e, E - 1), 0, 0)
    return pl.pallas_call(
        _expert_body,
        grid=(E + 1,),
        in_specs=[
            pl.BlockSpec((C, M), lambda e: (e, 0)),
            pl.BlockSpec((1, M, OUT), clamp),
            pl.BlockSpec((1, 1, M), clamp),
            pl.BlockSpec((1, 1, M), clamp),
            pl.BlockSpec((1, 1, OUT), clamp),
            pl.BlockSpec((C, 128), lambda e: (e, 0)),
        ],
        out_specs=pl.BlockSpec((C, OUT), lambda e: (e, 0)),
        out_shape=jax.ShapeDtypeStruct((NROWS, OUT), jnp.float32),
    )(xd, w_e, ln_g, ln_b, b_e, gslot)


def _combine_body(y_hbm, sd1_hbm, sd2_hbm, out_hbm,
                  rows1_v, rows2_v, idx1_v, idx2_v, sem1, sem2):
    wid = lax.axis_index("s") * NC + lax.axis_index("c")
    for half in range(TPW // CH):
        base = wid * TPW + half * CH
        pltpu.sync_copy(sd1_hbm.at[pl.ds(base, CH)], idx1_v)
        pltpu.sync_copy(sd2_hbm.at[pl.ds(base, CH)], idx2_v)
        cp1 = pltpu.async_copy(y_hbm.at[idx1_v], rows1_v, sem1)
        cp2 = pltpu.async_copy(y_hbm.at[idx2_v], rows2_v, sem2)
        cp1.wait()
        cp2.wait()

        def body_r(r, _):
            for j in range(OUT // 16):
                sl = pl.ds(j * 16, 16)
                rows1_v[r, sl] = rows1_v[r, sl] + rows2_v[r, sl]
            return 0

        lax.fori_loop(0, CH, body_r, 0)
        pltpu.sync_copy(rows1_v, out_hbm.at[pl.ds(base, CH)])


def _combine(y, sd1, sd2):
    mesh = plsc.VectorSubcoreMesh(core_axis_name="c", subcore_axis_name="s")
    f = functools.partial(
        pl.kernel,
        out_type=jax.ShapeDtypeStruct((S, OUT), jnp.float32),
        mesh=mesh,
        scratch_types=[
            pltpu.VMEM((CH, OUT), jnp.float32),
            pltpu.VMEM((CH, OUT), jnp.float32),
            pltpu.VMEM((CH,), jnp.int32),
            pltpu.VMEM((CH,), jnp.int32),
            pltpu.SemaphoreType.DMA,
            pltpu.SemaphoreType.DMA,
        ],
    )(_combine_body)
    return f(y, sd1, sd2)


def kernel(hidden_states, wg, w_e, b_e, ln_g, ln_b):
    x = hidden_states.reshape(S, M)
    sd1, sd2, g1, g2, laux = _gate(x, wg)
    xd, gslot = _dispatch(x, sd1, sd2, g1, g2)
    y = _expert(xd, w_e, ln_g.reshape(E, 1, M), ln_b.reshape(E, 1, M),
                b_e.reshape(E, 1, OUT), gslot)
    out = _combine(y, sd1, sd2)
    return out.reshape(B, T, OUT), laux.reshape(())


# speedup vs baseline: 1.6403x; 1.0494x over previous
import functools

import jax
import jax.numpy as jnp
from jax import lax
from jax.experimental import pallas as pl
from jax.experimental.pallas import tpu as pltpu
from jax.experimental.pallas import tpu_sc as plsc

B, T, M = 1, 2048, 1024
E = 8
OUT = 1024
S = B * T
C = 2 * S // E
NSLOT = E * C
NROWS = NSLOT + C
DUMMY = NSLOT

NC, NS = 2, 16
NW = NC * NS
TPW = S // NW
CH = 32
RB = 256
NBLK = S // RB


def _top2_masks_t(lt):
    iota_e = lax.broadcasted_iota(jnp.int32, lt.shape, 0)
    mx = jnp.max(lt, axis=0, keepdims=True)
    ex = jnp.exp(lt - mx)
    gates = ex / jnp.sum(ex, axis=0, keepdims=True)
    idx1 = jnp.min(jnp.where(lt == mx, iota_e, E), axis=0)
    mask1 = (iota_e == idx1[None, :]).astype(jnp.float32)
    masked = lt + (-1e9) * mask1
    m2 = jnp.max(masked, axis=0, keepdims=True)
    idx2 = jnp.min(jnp.where(masked == m2, iota_e, E), axis=0)
    mask2 = (iota_e == idx2[None, :]).astype(jnp.float32)
    return gates, idx1, mask1, idx2, mask2


def _gate_body(x_ref, wg_ref, sd1_ref, sd2_ref, g1_ref, g2_ref, laux_ref,
               log_s):
    i = pl.program_id(0)

    @pl.when(i < NBLK)
    def _():
        log_s[:, pl.ds(i * RB, RB)] = lax.dot_general(
            wg_ref[...], x_ref[...], (((0,), (1,)), ((), ())),
            preferred_element_type=jnp.float32)

    @pl.when(i >= NBLK)
    def _():
        ib = i - NBLK
        lt = log_s[...]
        gates_f, _, mask1_f, _, mask2_f = _top2_masks_t(lt)
        n1 = jnp.sum(mask1_f, axis=1, keepdims=True)
        laux_ref[...] = (jnp.sum(
            jnp.mean(gates_f, axis=1) * jnp.mean(mask1_f, axis=1)) * E
            ).reshape(1, 1)

        br = lax.broadcasted_iota(jnp.int32, (S, RB), 0)
        bc = lax.broadcasted_iota(jnp.int32, (S, RB), 1) + ib * RB
        tri = (bc > br).astype(jnp.bfloat16)
        locs1 = jnp.dot(mask1_f.astype(jnp.bfloat16), tri,
                        preferred_element_type=jnp.float32)
        locs2 = jnp.dot(mask2_f.astype(jnp.bfloat16), tri,
                        preferred_element_type=jnp.float32) + n1

        ltb = log_s[:, pl.ds(ib * RB, RB)]
        gates_b, idx1, mask1, idx2, mask2 = _top2_masks_t(ltb)
        mask1c = mask1 * (locs1 < C).astype(jnp.float32)
        mask2c = mask2 * (locs2 < C).astype(jnp.float32)
        loc1 = jnp.sum(locs1 * mask1c, axis=0).astype(jnp.int32)
        loc2 = jnp.sum(locs2 * mask2c, axis=0).astype(jnp.int32)
        gates1 = jnp.sum(gates_b * mask1c, axis=0)
        gates2 = jnp.sum(gates_b * mask2c, axis=0)
        denom = gates1 + gates2
        denom = jnp.where(denom < 1e-9, 1.0, denom)
        g1 = gates1 / denom
        g2 = gates2 / denom
        g1_ref[...] = jnp.broadcast_to(g1[:, None], (RB, 128))
        g2_ref[...] = jnp.broadcast_to(g2[:, None], (RB, 128))
        valid1 = jnp.sum(mask1c, axis=0) > 0
        valid2 = jnp.sum(mask2c, axis=0) > 0
        slot1 = idx1 * C + loc1
        slot2 = idx2 * C + loc2
        sd1_ref[...] = jnp.where(valid1, slot1, DUMMY).reshape(1, 1, RB)
        sd2_ref[...] = jnp.where(valid2, slot2, DUMMY).reshape(1, 1, RB)


def _gate(x, wg):
    islot = jax.ShapeDtypeStruct((NBLK, 1, RB), jnp.int32)
    fgate = jax.ShapeDtypeStruct((S, 128), jnp.float32)
    phase_b = lambda i: (jnp.maximum(i - NBLK, 0), 0, 0)
    phase_bg = lambda i: (jnp.maximum(i - NBLK, 0), 0)
    outs = pl.pallas_call(
        _gate_body,
        grid=(2 * NBLK,),
        in_specs=[
            pl.BlockSpec((RB, M), lambda i: (jnp.minimum(i, NBLK - 1), 0)),
            pl.BlockSpec((M, E), lambda i: (0, 0)),
        ],
        out_specs=[
            pl.BlockSpec((1, 1, RB), phase_b),
            pl.BlockSpec((1, 1, RB), phase_b),
            pl.BlockSpec((RB, 128), phase_bg),
            pl.BlockSpec((RB, 128), phase_bg),
            pl.BlockSpec((1, 1), lambda i: (0, 0)),
        ],
        out_shape=[islot, islot, fgate, fgate,
                   jax.ShapeDtypeStruct((1, 1), jnp.float32)],
        scratch_shapes=[pltpu.VMEM((E, S), jnp.float32)],
    )(x, wg)
    sd1, sd2, g1, g2, laux = outs
    return sd1.reshape(S), sd2.reshape(S), g1, g2, laux


DCH = TPW // 2


def _dispatch_body(x_hbm, sd1_hbm, sd2_hbm, xd_hbm,
                   rows0_v, rows1_v, i1a_v, i1b_v, i2a_v, i2b_v,
                   semr0, semr1, sems0, sems1):
    wid = lax.axis_index("s") * NC + lax.axis_index("c")
    base = wid * TPW
    rows = [rows0_v, rows1_v]
    i1 = [i1a_v, i1b_v]
    i2 = [i2a_v, i2b_v]
    semr = [semr0, semr1]
    sems = [sems0, sems1]
    lds = []
    for h in range(2):
        pltpu.sync_copy(sd1_hbm.at[pl.ds(base + h * DCH, DCH)], i1[h])
        pltpu.sync_copy(sd2_hbm.at[pl.ds(base + h * DCH, DCH)], i2[h])
        lds.append(pltpu.async_copy(
            x_hbm.at[pl.ds(base + h * DCH, DCH)], rows[h], semr[h]))
    scs = []
    for h in range(2):
        lds[h].wait()
        scs.append(pltpu.async_copy(rows[h], xd_hbm.at[i1[h]], sems[h]))
        scs.append(pltpu.async_copy(rows[h], xd_hbm.at[i2[h]], sems[h]))
    for c in scs:
        c.wait()


def _dispatch(x, sd1, sd2):
    mesh = plsc.VectorSubcoreMesh(core_axis_name="c", subcore_axis_name="s")
    f = functools.partial(
        pl.kernel,
        out_type=jax.ShapeDtypeStruct((NROWS, M), jnp.float32),
        mesh=mesh,
        scratch_types=[
            pltpu.VMEM((DCH, M), jnp.float32),
            pltpu.VMEM((DCH, M), jnp.float32),
            pltpu.VMEM((DCH,), jnp.int32),
            pltpu.VMEM((DCH,), jnp.int32),
            pltpu.VMEM((DCH,), jnp.int32),
            pltpu.VMEM((DCH,), jnp.int32),
            pltpu.SemaphoreType.DMA,
            pltpu.SemaphoreType.DMA,
            pltpu.SemaphoreType.DMA,
            pltpu.SemaphoreType.DMA,
        ],
    )(_dispatch_body)
    return f(x, sd1, sd2)


def _expert_body(xd_ref, we_ref, lng_ref, lnb_ref, be_ref, y_ref):
    e = pl.program_id(0)

    @pl.when(e < E)
    def _():
        x = xd_ref[...]
        mu = jnp.mean(x, axis=-1, keepdims=True)
        xc = x - mu
        var = jnp.mean(xc * xc, axis=-1, keepdims=True)
        xln = xc * lax.rsqrt(var + 1e-5)
        g = lng_ref[...].reshape(1, M)
        b = lnb_ref[...].reshape(1, M)
        normed = (xln * g + b).astype(jnp.bfloat16)
        w = we_ref[0].astype(jnp.bfloat16)
        y = jnp.dot(normed, w, preferred_element_type=jnp.float32)
        y = y + be_ref[...].reshape(1, OUT)
        y_ref[...] = jax.nn.gelu(y)

    @pl.when(e == E)
    def _():
        y_ref[...] = jnp.zeros((C, OUT), jnp.float32)


def _expert(xd, w_e, ln_g, ln_b, b_e):
    clamp = lambda e: (jnp.minimum(e, E - 1), 0, 0)
    return pl.pallas_call(
        _expert_body,
        grid=(E + 1,),
        in_specs=[
            pl.BlockSpec((C, M), lambda e: (e, 0)),
            pl.BlockSpec((1, M, OUT), clamp),
            pl.BlockSpec((1, 1, M), clamp),
            pl.BlockSpec((1, 1, M), clamp),
            pl.BlockSpec((1, 1, OUT), clamp),
        ],
        out_specs=pl.BlockSpec((C, OUT), lambda e: (e, 0)),
        out_shape=jax.ShapeDtypeStruct((NROWS, OUT), jnp.float32),
    )(xd, w_e, ln_g, ln_b, b_e)


CCH = 16
NCHC = TPW // CCH


def _combine_body(y_hbm, sd1_hbm, sd2_hbm, g1_hbm, g2_hbm, out_hbm,
                  r1a, r1b, r2a, r2b, oa, ob,
                  i1a, i1b, i2a, i2b, ga1, gb1, ga2, gb2,
                  semg0, semg1, semw0, semw1):
    wid = lax.axis_index("s") * NC + lax.axis_index("c")
    r1 = [r1a, r1b]
    r2 = [r2a, r2b]
    out = [oa, ob]
    i1 = [i1a, i1b]
    i2 = [i2a, i2b]
    g1 = [ga1, gb1]
    g2 = [ga2, gb2]
    semg = [semg0, semg1]
    semw = [semw0, semw1]
    gh = [None, None]
    wh = [None, None]

    def load_and_fire(h):
        b = h % 2
        base = wid * TPW + h * CCH
        pltpu.sync_copy(sd1_hbm.at[pl.ds(base, CCH)], i1[b])
        pltpu.sync_copy(sd2_hbm.at[pl.ds(base, CCH)], i2[b])
        pltpu.sync_copy(g1_hbm.at[pl.ds(base, CCH)], g1[b])
        pltpu.sync_copy(g2_hbm.at[pl.ds(base, CCH)], g2[b])
        gh[b] = (pltpu.async_copy(y_hbm.at[i1[b]], r1[b], semg[b]),
                 pltpu.async_copy(y_hbm.at[i2[b]], r2[b], semg[b]))

    load_and_fire(0)
    for h in range(NCHC):
        b = h % 2
        if h + 1 < NCHC:
            load_and_fire(h + 1)
        gh[b][0].wait()
        gh[b][1].wait()
        if wh[b] is not None:
            wh[b].wait()
        r1b_v, r2b_v, ob_v, g1b_v, g2b_v = r1[b], r2[b], out[b], g1[b], g2[b]

        def body_r(r, _):
            gw1 = g1b_v[r, pl.ds(0, 16)]
            gw2 = g2b_v[r, pl.ds(0, 16)]
            for j in range(OUT // 16):
                sl = pl.ds(j * 16, 16)
                ob_v[r, sl] = gw1 * r1b_v[r, sl] + gw2 * r2b_v[r, sl]
            return 0

        lax.fori_loop(0, CCH, body_r, 0)
        wh[b] = pltpu.async_copy(
            out[b], out_hbm.at[pl.ds(wid * TPW + h * CCH, CCH)], semw[b])
    for b in range(2):
        if wh[b] is not None:
            wh[b].wait()


def _combine(y, sd1, sd2, g1, g2):
    mesh = plsc.VectorSubcoreMesh(core_axis_name="c", subcore_axis_name="s")
    f = functools.partial(
        pl.kernel,
        out_type=jax.ShapeDtypeStruct((S, OUT), jnp.float32),
        mesh=mesh,
        scratch_types=(
            [pltpu.VMEM((CCH, OUT), jnp.float32)] * 6
            + [pltpu.VMEM((CCH,), jnp.int32)] * 4
            + [pltpu.VMEM((CCH, 128), jnp.float32)] * 4
            + [pltpu.SemaphoreType.DMA] * 4
        ),
    )(_combine_body)
    return f(y, sd1, sd2, g1, g2)


def kernel(hidden_states, wg, w_e, b_e, ln_g, ln_b):
    x = hidden_states.reshape(S, M)
    sd1, sd2, g1, g2, laux = _gate(x, wg)
    xd = _dispatch(x, sd1, sd2)
    y = _expert(xd, w_e, ln_g.reshape(E, 1, M), ln_b.reshape(E, 1, M),
                b_e.reshape(E, 1, OUT))
    out = _combine(y, sd1, sd2, g1, g2)
    return out.reshape(B, T, OUT), laux.reshape(())


# speedup vs baseline: 1.7005x; 1.0367x over previous
import functools

import jax
import jax.numpy as jnp
from jax import lax
from jax.experimental import pallas as pl
from jax.experimental.pallas import tpu as pltpu
from jax.experimental.pallas import tpu_sc as plsc

B, T, M = 1, 2048, 1024
E = 8
OUT = 1024
S = B * T
C = 2 * S // E
NSLOT = E * C
NROWS = NSLOT + C
DUMMY = NSLOT

NC, NS = 2, 16
NW = NC * NS
TPW = S // NW
CH = 32
RB = 256
NBLK = S // RB


def _top2_masks_t(lt):
    iota_e = lax.broadcasted_iota(jnp.int32, lt.shape, 0)
    mx = jnp.max(lt, axis=0, keepdims=True)
    ex = jnp.exp(lt - mx)
    gates = ex / jnp.sum(ex, axis=0, keepdims=True)
    idx1 = jnp.min(jnp.where(lt == mx, iota_e, E), axis=0)
    mask1 = (iota_e == idx1[None, :]).astype(jnp.float32)
    masked = lt + (-1e9) * mask1
    m2 = jnp.max(masked, axis=0, keepdims=True)
    idx2 = jnp.min(jnp.where(masked == m2, iota_e, E), axis=0)
    mask2 = (iota_e == idx2[None, :]).astype(jnp.float32)
    return gates, idx1, mask1, idx2, mask2


def _gate_body(x_ref, wg_ref, sd1_ref, sd2_ref, g1_ref, g2_ref, laux_ref,
               log_s, n1_s, carry_s):
    i = pl.program_id(0)

    @pl.when(i < NBLK)
    def _():
        log_s[:, pl.ds(i * RB, RB)] = lax.dot_general(
            wg_ref[...], x_ref[...], (((0,), (1,)), ((), ())),
            preferred_element_type=jnp.float32)

    @pl.when(i == NBLK)
    def _():
        lt = log_s[...]
        gates_f, _, mask1_f, _, _ = _top2_masks_t(lt)
        n1_s[...] = jnp.broadcast_to(
            jnp.sum(mask1_f, axis=1, keepdims=True), (E, 128))
        carry_s[...] = jnp.zeros((2 * E, 128), jnp.float32)
        laux_ref[...] = (jnp.sum(
            jnp.mean(gates_f, axis=1) * jnp.mean(mask1_f, axis=1)) * E
            ).reshape(1, 1)

    @pl.when(i >= NBLK)
    def _():
        ib = i - NBLK
        ltb = log_s[:, pl.ds(ib * RB, RB)]
        gates_b, idx1, mask1, idx2, mask2 = _top2_masks_t(ltb)

        br = lax.broadcasted_iota(jnp.int32, (RB, RB), 0)
        bc = lax.broadcasted_iota(jnp.int32, (RB, RB), 1)
        tri = (bc > br).astype(jnp.bfloat16)
        n1 = n1_s[:, 0:1]
        c1 = carry_s[0:E, 0:1]
        c2 = carry_s[E:2 * E, 0:1]
        locs1 = jnp.dot(mask1.astype(jnp.bfloat16), tri,
                        preferred_element_type=jnp.float32) + c1
        locs2 = jnp.dot(mask2.astype(jnp.bfloat16), tri,
                        preferred_element_type=jnp.float32) + c2 + n1
        carry_s[0:E, :] = jnp.broadcast_to(
            c1 + jnp.sum(mask1, axis=1, keepdims=True), (E, 128))
        carry_s[E:2 * E, :] = jnp.broadcast_to(
            c2 + jnp.sum(mask2, axis=1, keepdims=True), (E, 128))
        mask1c = mask1 * (locs1 < C).astype(jnp.float32)
        mask2c = mask2 * (locs2 < C).astype(jnp.float32)
        loc1 = jnp.sum(locs1 * mask1c, axis=0).astype(jnp.int32)
        loc2 = jnp.sum(locs2 * mask2c, axis=0).astype(jnp.int32)
        gates1 = jnp.sum(gates_b * mask1c, axis=0)
        gates2 = jnp.sum(gates_b * mask2c, axis=0)
        denom = gates1 + gates2
        denom = jnp.where(denom < 1e-9, 1.0, denom)
        g1 = gates1 / denom
        g2 = gates2 / denom
        g1_ref[...] = jnp.broadcast_to(g1[:, None], (RB, 128))
        g2_ref[...] = jnp.broadcast_to(g2[:, None], (RB, 128))
        valid1 = jnp.sum(mask1c, axis=0) > 0
        valid2 = jnp.sum(mask2c, axis=0) > 0
        slot1 = idx1 * C + loc1
        slot2 = idx2 * C + loc2
        sd1_ref[...] = jnp.where(valid1, slot1, DUMMY).reshape(1, 1, RB)
        sd2_ref[...] = jnp.where(valid2, slot2, DUMMY).reshape(1, 1, RB)


def _gate(x, wg):
    islot = jax.ShapeDtypeStruct((NBLK, 1, RB), jnp.int32)
    fgate = jax.ShapeDtypeStruct((S, 128), jnp.float32)
    phase_b = lambda i: (jnp.maximum(i - NBLK, 0), 0, 0)
    phase_bg = lambda i: (jnp.maximum(i - NBLK, 0), 0)
    outs = pl.pallas_call(
        _gate_body,
        grid=(2 * NBLK,),
        in_specs=[
            pl.BlockSpec((RB, M), lambda i: (jnp.minimum(i, NBLK - 1), 0)),
            pl.BlockSpec((M, E), lambda i: (0, 0)),
        ],
        out_specs=[
            pl.BlockSpec((1, 1, RB), phase_b),
            pl.BlockSpec((1, 1, RB), phase_b),
            pl.BlockSpec((RB, 128), phase_bg),
            pl.BlockSpec((RB, 128), phase_bg),
            pl.BlockSpec((1, 1), lambda i: (0, 0)),
        ],
        out_shape=[islot, islot, fgate, fgate,
                   jax.ShapeDtypeStruct((1, 1), jnp.float32)],
        scratch_shapes=[pltpu.VMEM((E, S), jnp.float32),
                        pltpu.VMEM((E, 128), jnp.float32),
                        pltpu.VMEM((2 * E, 128), jnp.float32)],
    )(x, wg)
    sd1, sd2, g1, g2, laux = outs
    return sd1.reshape(S), sd2.reshape(S), g1, g2, laux


DCH = TPW // 2


def _dispatch_body(x_hbm, sd1_hbm, sd2_hbm, xd_hbm,
                   rows0_v, rows1_v, i1a_v, i1b_v, i2a_v, i2b_v,
                   semr0, semr1, sems0, sems1):
    wid = lax.axis_index("s") * NC + lax.axis_index("c")
    base = wid * TPW
    rows = [rows0_v, rows1_v]
    i1 = [i1a_v, i1b_v]
    i2 = [i2a_v, i2b_v]
    semr = [semr0, semr1]
    sems = [sems0, sems1]
    lds = []
    for h in range(2):
        pltpu.sync_copy(sd1_hbm.at[pl.ds(base + h * DCH, DCH)], i1[h])
        pltpu.sync_copy(sd2_hbm.at[pl.ds(base + h * DCH, DCH)], i2[h])
        lds.append(pltpu.async_copy(
            x_hbm.at[pl.ds(base + h * DCH, DCH)], rows[h], semr[h]))
    scs = []
    for h in range(2):
        lds[h].wait()
        scs.append(pltpu.async_copy(rows[h], xd_hbm.at[i1[h]], sems[h]))
        scs.append(pltpu.async_copy(rows[h], xd_hbm.at[i2[h]], sems[h]))
    for c in scs:
        c.wait()


def _dispatch(x, sd1, sd2):
    mesh = plsc.VectorSubcoreMesh(core_axis_name="c", subcore_axis_name="s")
    f = functools.partial(
        pl.kernel,
        out_type=jax.ShapeDtypeStruct((NROWS, M), jnp.float32),
        mesh=mesh,
        scratch_types=[
            pltpu.VMEM((DCH, M), jnp.float32),
            pltpu.VMEM((DCH, M), jnp.float32),
            pltpu.VMEM((DCH,), jnp.int32),
            pltpu.VMEM((DCH,), jnp.int32),
            pltpu.VMEM((DCH,), jnp.int32),
            pltpu.VMEM((DCH,), jnp.int32),
            pltpu.SemaphoreType.DMA,
            pltpu.SemaphoreType.DMA,
            pltpu.SemaphoreType.DMA,
            pltpu.SemaphoreType.DMA,
        ],
    )(_dispatch_body)
    return f(x, sd1, sd2)


def _expert_body(xd_ref, we_ref, lng_ref, lnb_ref, be_ref, y_ref):
    e = pl.program_id(0)

    @pl.when(e < E)
    def _():
        x = xd_ref[...]
        mu = jnp.mean(x, axis=-1, keepdims=True)
        xc = x - mu
        var = jnp.mean(xc * xc, axis=-1, keepdims=True)
        xln = xc * lax.rsqrt(var + 1e-5)
        g = lng_ref[...].reshape(1, M)
        b = lnb_ref[...].reshape(1, M)
        normed = (xln * g + b).astype(jnp.bfloat16)
        w = we_ref[0].astype(jnp.bfloat16)
        y = jnp.dot(normed, w, preferred_element_type=jnp.float32)
        y = y + be_ref[...].reshape(1, OUT)
        y_ref[...] = jax.nn.gelu(y)

    @pl.when(e == E)
    def _():
        y_ref[...] = jnp.zeros((C, OUT), jnp.float32)


def _expert(xd, w_e, ln_g, ln_b, b_e):
    clamp = lambda e: (jnp.minimum(e, E - 1), 0, 0)
    return pl.pallas_call(
        _expert_body,
        grid=(E + 1,),
        in_specs=[
            pl.BlockSpec((C, M), lambda e: (e, 0)),
            pl.BlockSpec((1, M, OUT), clamp),
            pl.BlockSpec((1, 1, M), clamp),
            pl.BlockSpec((1, 1, M), clamp),
            pl.BlockSpec((1, 1, OUT), clamp),
        ],
        out_specs=pl.BlockSpec((C, OUT), lambda e: (e, 0)),
        out_shape=jax.ShapeDtypeStruct((NROWS, OUT), jnp.float32),
    )(xd, w_e, ln_g, ln_b, b_e)


CCH = 16
NCHC = TPW // CCH


def _combine_body(y_hbm, sd1_hbm, sd2_hbm, g1_hbm, g2_hbm, out_hbm,
                  r1a, r1b, r2a, r2b, oa, ob,
                  i1c0, i1c1, i1c2, i1c3, i2c0, i2c1, i2c2, i2c3,
                  g1_all, g2_all,
                  semi, semg0, semg1, semw0, semw1):
    wid = lax.axis_index("s") * NC + lax.axis_index("c")
    base = wid * TPW
    r1 = [r1a, r1b]
    r2 = [r2a, r2b]
    out = [oa, ob]
    i1 = [i1c0, i1c1, i1c2, i1c3]
    i2 = [i2c0, i2c1, i2c2, i2c3]
    semg = [semg0, semg1]
    semw = [semw0, semw1]

    lds = []
    for h in range(NCHC):
        lds.append(pltpu.async_copy(
            sd1_hbm.at[pl.ds(base + h * CCH, CCH)], i1[h], semi))
        lds.append(pltpu.async_copy(
            sd2_hbm.at[pl.ds(base + h * CCH, CCH)], i2[h], semi))
    lds.append(pltpu.async_copy(g1_hbm.at[pl.ds(base, TPW)], g1_all, semi))
    lds.append(pltpu.async_copy(g2_hbm.at[pl.ds(base, TPW)], g2_all, semi))
    for c in lds:
        c.wait()

    gh = [None, None]
    wh = [None, None]

    def fire(h):
        b = h % 2
        gh[b] = (pltpu.async_copy(y_hbm.at[i1[h]], r1[b], semg[b]),
                 pltpu.async_copy(y_hbm.at[i2[h]], r2[b], semg[b]))

    fire(0)
    for h in range(NCHC):
        b = h % 2
        if h + 1 < NCHC:
            fire(h + 1)
        gh[b][0].wait()
        gh[b][1].wait()
        if wh[b] is not None:
            wh[b].wait()
        r1b_v, r2b_v, ob_v = r1[b], r2[b], out[b]
        off = h * CCH

        def body_r(r, _):
            gw1 = g1_all[off + r, pl.ds(0, 16)]
            gw2 = g2_all[off + r, pl.ds(0, 16)]
            for j in range(OUT // 16):
                sl = pl.ds(j * 16, 16)
                ob_v[r, sl] = gw1 * r1b_v[r, sl] + gw2 * r2b_v[r, sl]
            return 0

        lax.fori_loop(0, CCH, body_r, 0)
        wh[b] = pltpu.async_copy(
            out[b], out_hbm.at[pl.ds(base + h * CCH, CCH)], semw[b])
    for b in range(2):
        if wh[b] is not None:
            wh[b].wait()


def _combine(y, sd1, sd2, g1, g2):
    mesh = plsc.VectorSubcoreMesh(core_axis_name="c", subcore_axis_name="s")
    f = functools.partial(
        pl.kernel,
        out_type=jax.ShapeDtypeStruct((S, OUT), jnp.float32),
        mesh=mesh,
        scratch_types=(
            [pltpu.VMEM((CCH, OUT), jnp.float32)] * 6
            + [pltpu.VMEM((CCH,), jnp.int32)] * 8
            + [pltpu.VMEM((TPW, 128), jnp.float32)] * 2
            + [pltpu.SemaphoreType.DMA] * 5
        ),
    )(_combine_body)
    return f(y, sd1, sd2, g1, g2)


def kernel(hidden_states, wg, w_e, b_e, ln_g, ln_b):
    x = hidden_states.reshape(S, M)
    sd1, sd2, g1, g2, laux = _gate(x, wg)
    xd = _dispatch(x, sd1, sd2)
    y = _expert(xd, w_e, ln_g.reshape(E, 1, M), ln_b.reshape(E, 1, M),
                b_e.reshape(E, 1, OUT))
    out = _combine(y, sd1, sd2, g1, g2)
    return out.reshape(B, T, OUT), laux.reshape(())


# speedup vs baseline: 1.7453x; 1.0264x over previous
import functools

import jax
import jax.numpy as jnp
from jax import lax
from jax.experimental import pallas as pl
from jax.experimental.pallas import tpu as pltpu
from jax.experimental.pallas import tpu_sc as plsc

B, T, M = 1, 2048, 1024
E = 8
OUT = 1024
S = B * T
C = 2 * S // E
NSLOT = E * C
NROWS = NSLOT + C
DUMMY = NSLOT

NC, NS = 2, 16
NW = NC * NS
TPW = S // NW
CH = 32
RB = 256
NBLK = S // RB
RBA = 512
NBLKA = S // RBA


def _top2_masks_t(lt):
    iota_e = lax.broadcasted_iota(jnp.int32, lt.shape, 0)
    mx = jnp.max(lt, axis=0, keepdims=True)
    ex = jnp.exp(lt - mx)
    gates = ex / jnp.sum(ex, axis=0, keepdims=True)
    idx1 = jnp.min(jnp.where(lt == mx, iota_e, E), axis=0)
    mask1 = (iota_e == idx1[None, :]).astype(jnp.float32)
    masked = lt + (-1e9) * mask1
    m2 = jnp.max(masked, axis=0, keepdims=True)
    idx2 = jnp.min(jnp.where(masked == m2, iota_e, E), axis=0)
    mask2 = (iota_e == idx2[None, :]).astype(jnp.float32)
    return gates, idx1, mask1, idx2, mask2


def _gate_body(x_ref, wg_ref, sd1_ref, sd2_ref, g1_ref, g2_ref, laux_ref,
               log_s, n1_s, carry_s):
    i = pl.program_id(0)

    @pl.when(i < NBLKA)
    def _():
        log_s[:, pl.ds(i * RBA, RBA)] = lax.dot_general(
            wg_ref[...], x_ref[...], (((0,), (1,)), ((), ())),
            preferred_element_type=jnp.float32)

    @pl.when(i == NBLKA)
    def _():
        lt = log_s[...]
        gates_f, _, mask1_f, _, _ = _top2_masks_t(lt)
        n1_s[...] = jnp.broadcast_to(
            jnp.sum(mask1_f, axis=1, keepdims=True), (E, 128))
        carry_s[...] = jnp.zeros((2 * E, 128), jnp.float32)
        laux_ref[...] = (jnp.sum(
            jnp.mean(gates_f, axis=1) * jnp.mean(mask1_f, axis=1)) * E
            ).reshape(1, 1)

    @pl.when(i >= NBLKA)
    def _():
        ib = i - NBLKA
        ltb = log_s[:, pl.ds(ib * RB, RB)]
        gates_b, idx1, mask1, idx2, mask2 = _top2_masks_t(ltb)

        br = lax.broadcasted_iota(jnp.int32, (RB, RB), 0)
        bc = lax.broadcasted_iota(jnp.int32, (RB, RB), 1)
        tri = (bc > br).astype(jnp.bfloat16)
        n1 = n1_s[:, 0:1]
        c1 = carry_s[0:E, 0:1]
        c2 = carry_s[E:2 * E, 0:1]
        locs1 = jnp.dot(mask1.astype(jnp.bfloat16), tri,
                        preferred_element_type=jnp.float32) + c1
        locs2 = jnp.dot(mask2.astype(jnp.bfloat16), tri,
                        preferred_element_type=jnp.float32) + c2 + n1
        carry_s[0:E, :] = jnp.broadcast_to(
            c1 + jnp.sum(mask1, axis=1, keepdims=True), (E, 128))
        carry_s[E:2 * E, :] = jnp.broadcast_to(
            c2 + jnp.sum(mask2, axis=1, keepdims=True), (E, 128))
        mask1c = mask1 * (locs1 < C).astype(jnp.float32)
        mask2c = mask2 * (locs2 < C).astype(jnp.float32)
        loc1 = jnp.sum(locs1 * mask1c, axis=0).astype(jnp.int32)
        loc2 = jnp.sum(locs2 * mask2c, axis=0).astype(jnp.int32)
        gates1 = jnp.sum(gates_b * mask1c, axis=0)
        gates2 = jnp.sum(gates_b * mask2c, axis=0)
        denom = gates1 + gates2
        denom = jnp.where(denom < 1e-9, 1.0, denom)
        g1 = gates1 / denom
        g2 = gates2 / denom
        g1_ref[...] = jnp.broadcast_to(g1[:, None], (RB, 128))
        g2_ref[...] = jnp.broadcast_to(g2[:, None], (RB, 128))
        valid1 = jnp.sum(mask1c, axis=0) > 0
        valid2 = jnp.sum(mask2c, axis=0) > 0
        slot1 = idx1 * C + loc1
        slot2 = idx2 * C + loc2
        nr = RB // 128
        sd1_ref[pl.ds(ib * nr, nr), :] = jnp.where(
            valid1, slot1, DUMMY).reshape(nr, 128)
        sd2_ref[pl.ds(ib * nr, nr), :] = jnp.where(
            valid2, slot2, DUMMY).reshape(nr, 128)


def _gate(x, wg):
    islot = jax.ShapeDtypeStruct((S // 128, 128), jnp.int32)
    fgate = jax.ShapeDtypeStruct((S, 128), jnp.float32)
    phase_b = lambda i: (jnp.maximum(i - NBLKA, 0), 0)
    outs = pl.pallas_call(
        _gate_body,
        grid=(NBLKA + NBLK,),
        in_specs=[
            pl.BlockSpec((RBA, M), lambda i: (jnp.minimum(i, NBLKA - 1), 0)),
            pl.BlockSpec((M, E), lambda i: (0, 0)),
        ],
        out_specs=[
            pl.BlockSpec((S // 128, 128), lambda i: (0, 0)),
            pl.BlockSpec((S // 128, 128), lambda i: (0, 0)),
            pl.BlockSpec((RB, 128), phase_b),
            pl.BlockSpec((RB, 128), phase_b),
            pl.BlockSpec((1, 1), lambda i: (0, 0)),
        ],
        out_shape=[islot, islot, fgate, fgate,
                   jax.ShapeDtypeStruct((1, 1), jnp.float32)],
        scratch_shapes=[pltpu.VMEM((E, S), jnp.float32),
                        pltpu.VMEM((E, 128), jnp.float32),
                        pltpu.VMEM((2 * E, 128), jnp.float32)],
    )(x, wg)
    sd1, sd2, g1, g2, laux = outs
    return sd1.reshape(S), sd2.reshape(S), g1, g2, laux


DCH = TPW // 2


def _dispatch_body(x_hbm, sd1_hbm, sd2_hbm, xd_hbm,
                   rows0_v, rows1_v, i1a_v, i1b_v, i2a_v, i2b_v,
                   semr0, semr1, sems0, sems1):
    wid = lax.axis_index("s") * NC + lax.axis_index("c")
    base = wid * TPW
    rows = [rows0_v, rows1_v]
    i1 = [i1a_v, i1b_v]
    i2 = [i2a_v, i2b_v]
    semr = [semr0, semr1]
    sems = [sems0, sems1]
    lds = []
    for h in range(2):
        pltpu.sync_copy(sd1_hbm.at[pl.ds(base + h * DCH, DCH)], i1[h])
        pltpu.sync_copy(sd2_hbm.at[pl.ds(base + h * DCH, DCH)], i2[h])
        lds.append(pltpu.async_copy(
            x_hbm.at[pl.ds(base + h * DCH, DCH)], rows[h], semr[h]))
    scs = []
    for h in range(2):
        lds[h].wait()
        scs.append(pltpu.async_copy(rows[h], xd_hbm.at[i1[h]], sems[h]))
        scs.append(pltpu.async_copy(rows[h], xd_hbm.at[i2[h]], sems[h]))
    for c in scs:
        c.wait()


def _dispatch(x, sd1, sd2):
    mesh = plsc.VectorSubcoreMesh(core_axis_name="c", subcore_axis_name="s")
    f = functools.partial(
        pl.kernel,
        out_type=jax.ShapeDtypeStruct((NROWS, M), jnp.float32),
        mesh=mesh,
        scratch_types=[
            pltpu.VMEM((DCH, M), jnp.float32),
            pltpu.VMEM((DCH, M), jnp.float32),
            pltpu.VMEM((DCH,), jnp.int32),
            pltpu.VMEM((DCH,), jnp.int32),
            pltpu.VMEM((DCH,), jnp.int32),
            pltpu.VMEM((DCH,), jnp.int32),
            pltpu.SemaphoreType.DMA,
            pltpu.SemaphoreType.DMA,
            pltpu.SemaphoreType.DMA,
            pltpu.SemaphoreType.DMA,
        ],
    )(_dispatch_body)
    return f(x, sd1, sd2)


def _expert_body(xd_ref, we_ref, lng_ref, lnb_ref, be_ref, y_ref):
    e = pl.program_id(0)

    @pl.when(e < E)
    def _():
        x = xd_ref[...]
        mu = jnp.mean(x, axis=-1, keepdims=True)
        xc = x - mu
        var = jnp.mean(xc * xc, axis=-1, keepdims=True)
        xln = xc * lax.rsqrt(var + 1e-5)
        g = lng_ref[...].reshape(1, M)
        b = lnb_ref[...].reshape(1, M)
        normed = (xln * g + b).astype(jnp.bfloat16)
        w = we_ref[0].astype(jnp.bfloat16)
        y = jnp.dot(normed, w, preferred_element_type=jnp.float32)
        y = y + be_ref[...].reshape(1, OUT)
        y_ref[...] = jax.nn.gelu(y)

    @pl.when(e == E)
    def _():
        y_ref[...] = jnp.zeros((C, OUT), jnp.float32)


def _expert(xd, w_e, ln_g, ln_b, b_e):
    clamp = lambda e: (jnp.minimum(e, E - 1), 0, 0)
    return pl.pallas_call(
        _expert_body,
        grid=(E + 1,),
        in_specs=[
            pl.BlockSpec((C, M), lambda e: (e, 0)),
            pl.BlockSpec((1, M, OUT), clamp),
            pl.BlockSpec((1, 1, M), clamp),
            pl.BlockSpec((1, 1, M), clamp),
            pl.BlockSpec((1, 1, OUT), clamp),
        ],
        out_specs=pl.BlockSpec((C, OUT), lambda e: (e, 0)),
        out_shape=jax.ShapeDtypeStruct((NROWS, OUT), jnp.float32),
    )(xd, w_e, ln_g, ln_b, b_e)


CCH = 16
NCHC = TPW // CCH


def _combine_body(y_hbm, sd1_hbm, sd2_hbm, g1_hbm, g2_hbm, out_hbm,
                  r1a, r1b, r2a, r2b, oa, ob,
                  i1c0, i1c1, i1c2, i1c3, i2c0, i2c1, i2c2, i2c3,
                  g1_all, g2_all,
                  semi, semg0, semg1, semw0, semw1):
    wid = lax.axis_index("s") * NC + lax.axis_index("c")
    base = wid * TPW
    r1 = [r1a, r1b]
    r2 = [r2a, r2b]
    out = [oa, ob]
    i1 = [i1c0, i1c1, i1c2, i1c3]
    i2 = [i2c0, i2c1, i2c2, i2c3]
    semg = [semg0, semg1]
    semw = [semw0, semw1]

    lds = []
    for h in range(NCHC):
        lds.append(pltpu.async_copy(
            sd1_hbm.at[pl.ds(base + h * CCH, CCH)], i1[h], semi))
        lds.append(pltpu.async_copy(
            sd2_hbm.at[pl.ds(base + h * CCH, CCH)], i2[h], semi))
    lds.append(pltpu.async_copy(g1_hbm.at[pl.ds(base, TPW)], g1_all, semi))
    lds.append(pltpu.async_copy(g2_hbm.at[pl.ds(base, TPW)], g2_all, semi))
    for c in lds:
        c.wait()

    gh = [None, None]
    wh = [None, None]

    def fire(h):
        b = h % 2
        gh[b] = (pltpu.async_copy(y_hbm.at[i1[h]], r1[b], semg[b]),
                 pltpu.async_copy(y_hbm.at[i2[h]], r2[b], semg[b]))

    fire(0)
    for h in range(NCHC):
        b = h % 2
        if h + 1 < NCHC:
            fire(h + 1)
        gh[b][0].wait()
        gh[b][1].wait()
        if wh[b] is not None:
            wh[b].wait()
        r1b_v, r2b_v, ob_v = r1[b], r2[b], out[b]
        off = h * CCH

        def body_r(r, _):
            gw1 = g1_all[off + r, pl.ds(0, 16)]
            gw2 = g2_all[off + r, pl.ds(0, 16)]
            for j in range(OUT // 16):
                sl = pl.ds(j * 16, 16)
                ob_v[r, sl] = gw1 * r1b_v[r, sl] + gw2 * r2b_v[r, sl]
            return 0

        lax.fori_loop(0, CCH, body_r, 0)
        wh[b] = pltpu.async_copy(
            out[b], out_hbm.at[pl.ds(base + h * CCH, CCH)], semw[b])
    for b in range(2):
        if wh[b] is not None:
            wh[b].wait()


def _combine(y, sd1, sd2, g1, g2):
    mesh = plsc.VectorSubcoreMesh(core_axis_name="c", subcore_axis_name="s")
    f = functools.partial(
        pl.kernel,
        out_type=jax.ShapeDtypeStruct((S, OUT), jnp.float32),
        mesh=mesh,
        scratch_types=(
            [pltpu.VMEM((CCH, OUT), jnp.float32)] * 6
            + [pltpu.VMEM((CCH,), jnp.int32)] * 8
            + [pltpu.VMEM((TPW, 128), jnp.float32)] * 2
            + [pltpu.SemaphoreType.DMA] * 5
        ),
    )(_combine_body)
    return f(y, sd1, sd2, g1, g2)


def kernel(hidden_states, wg, w_e, b_e, ln_g, ln_b):
    x = hidden_states.reshape(S, M)
    sd1, sd2, g1, g2, laux = _gate(x, wg)
    xd = _dispatch(x, sd1, sd2)
    y = _expert(xd, w_e, ln_g.reshape(E, 1, M), ln_b.reshape(E, 1, M),
                b_e.reshape(E, 1, OUT))
    out = _combine(y, sd1, sd2, g1, g2)
    return out.reshape(B, T, OUT), laux.reshape(())
